# baseline (speedup 1.0000x reference)
import functools

import jax
import jax.numpy as jnp
from jax import lax
from jax.experimental import pallas as pl
from jax.experimental.pallas import tpu as pltpu

_N = 32
_C = 16
_KD = 8
_IMG = 16
_H = 2
_BT = 8
_GROUPS = 8
_BSTEP = _BT * _GROUPS

_VRAW = 48
_AGOFF = 80


def _body(x_ref, wbig_ref, bbig_ref, rep2_ref, wexp_ref, shift2_ref,
          bd_ref, biasT_ref, mask_ref, o_ref):
    f32 = jnp.float32
    bf16 = jnp.bfloat16
    x = x_ref[...].astype(bf16)
    qkv = jnp.dot(x, wbig_ref[...],
                  preferred_element_type=f32).astype(bf16) + bbig_ref[...]

    rows = _BT * _N
    rows_all = _BSTEP * _N
    mask = mask_ref[...]
    xTa = jnp.concatenate([x.T, jnp.ones((1, rows_all), bf16),
                           jnp.zeros((7, rows_all), bf16)],
                          axis=0)

    zs = []
    for g in range(_GROUPS):
        r0 = g * rows
        zg = None
        for h in range(_H):
            ag = qkv[r0:r0 + rows, _AGOFF + 24 * h:_AGOFF + 24 * h + 24]
            va = qkv[r0:r0 + rows, 16 * h:16 * h + 32]
            s = jnp.dot(ag, xTa[:, r0:r0 + rows],
                        preferred_element_type=f32)
            p = jnp.exp(s).astype(bf16) * mask
            oa = jnp.dot(p, va, preferred_element_type=f32)
            if h == 0:
                t = oa[:, :_IMG] * pl.reciprocal(oa[:, _IMG:_IMG + 1],
                                                 approx=True)
            else:
                t = oa[:, _IMG:] * pl.reciprocal(oa[:, 0:1], approx=True)
            zg = t if zg is None else zg + t
        zs.append(zg)

    v0 = qkv[:, _VRAW:_VRAW + 32].reshape(_BSTEP, _N, 32)[:, :_IMG, :]
    v0 = v0.reshape(_BSTEP * _IMG, 32)
    v0 = v0 * jnp.clip(v0 + 3.0, 0.0, 6.0)
    lhs = jnp.dot(v0, rep2_ref[...], preferred_element_type=f32)
    lhs = lhs.astype(bf16) * wexp_ref[...]
    conv = jnp.dot(lhs, shift2_ref[...],
                   preferred_element_type=f32)

    outs = []
    for g in range(_GROUPS):
        cg = conv[g * _BT * _IMG:(g + 1) * _BT * _IMG]
        cat = jnp.concatenate([zs[g], cg[:, :_IMG], cg[:, _IMG:]],
                              axis=0).astype(bf16)
        outs.append(jnp.dot(bd_ref[...], cat,
                            preferred_element_type=f32))
    outT = jnp.concatenate(outs, axis=0) + biasT_ref[...]
    o_ref[...] = outT.reshape(_BSTEP, _C, _IMG).transpose(0, 2, 1)


@jax.jit
def kernel(x, w_q, w_k, w_v, b_q, b_k, b_v, w_proj, w_exp, rep_mat,
           shift_stack, w_out, out_bias):
    B, N, C = x.shape
    f32 = jnp.float32
    bf16 = jnp.bfloat16

    wv0p = w_v[0] @ w_proj[0]
    wv1p = w_v[1] @ w_proj[1]
    bv0p = (b_v[0] @ w_proj[0])[0]
    bv1p = (b_v[1] @ w_proj[1])[0]
    one16 = jnp.ones((16,), f32)

    wags, bags = [], []
    for h in range(_H):
        wag = jnp.concatenate([w_q[h] @ w_k[h].T,
                               (w_q[h] @ b_k[h, 0])[:, None],
                               jnp.zeros((C, 7), f32)], axis=1)
        bag = jnp.concatenate([b_q[h, 0] @ w_k[h].T,
                               jnp.sum(b_q[h, 0] * b_k[h, 0])[None],
                               jnp.zeros((7,), f32)])
        wags.append(wag)
        bags.append(bag)

    wbig = jnp.concatenate([wv0p, jnp.zeros((C, 16), f32), wv1p,
                            w_v[0], w_v[1], wags[0], wags[1]],
                           axis=1).astype(bf16)
    bbig = jnp.concatenate([bv0p, one16, bv1p, b_v[0, 0], b_v[1, 0],
                            bags[0], bags[1]])[None, :].astype(bf16)

    eye2 = jnp.eye(2, dtype=f32)
    rep2 = jnp.kron(eye2, rep_mat).astype(bf16)
    shift2 = jnp.kron(eye2, shift_stack).astype(bf16)
    wexp = jnp.tile(jnp.concatenate([w_exp[0], w_exp[1]], axis=1) * (1.0 / 6.0),
                    (_BSTEP, 1)).astype(bf16)

    woutT = w_out.T
    eyeb = jnp.eye(_BT, dtype=f32)
    bd = jnp.concatenate([jnp.kron(eyeb, woutT),
                          jnp.kron(eyeb, woutT[:, :_IMG]),
                          jnp.kron(eyeb, woutT[:, _IMG:])],
                         axis=1).astype(bf16)
    biasT = jnp.tile(out_bias.T, (_BSTEP, 1))

    rows = _BT * _N
    bi = jnp.arange(rows, dtype=jnp.int32) // _N
    mask = (bi[:, None] == bi[None, :]).astype(bf16)

    x2 = x.reshape(B * N, C)
    steps = B // _BSTEP
    const = lambda g: (0, 0)
    out = pl.pallas_call(
        _body,
        out_shape=jax.ShapeDtypeStruct((B, _IMG, _C), f32),
        grid=(steps,),
        in_specs=[
            pl.BlockSpec((_BSTEP * _N, C), lambda g: (g, 0)),
            pl.BlockSpec(wbig.shape, const),
            pl.BlockSpec(bbig.shape, const),
            pl.BlockSpec(rep2.shape, const),
            pl.BlockSpec(wexp.shape, const),
            pl.BlockSpec(shift2.shape, const),
            pl.BlockSpec(bd.shape, const),
            pl.BlockSpec(biasT.shape, const),
            pl.BlockSpec(mask.shape, const),
        ],
        out_specs=pl.BlockSpec((_BSTEP, _IMG, _C), lambda g: (g, 0, 0)),
        compiler_params=pltpu.CompilerParams(
            dimension_semantics=("parallel",)),
    )(x2, wbig, bbig, rep2, wexp, shift2, bd, biasT, mask)
    return out


# speedup vs baseline: 1.0238x; 1.0238x over previous
import functools

import jax
import jax.numpy as jnp
from jax import lax
from jax.experimental import pallas as pl
from jax.experimental.pallas import tpu as pltpu

_N = 32
_C = 16
_KD = 8
_IMG = 16
_H = 2
_BT = 8
_GROUPS = 8
_BSTEP = _BT * _GROUPS

_VRAW = 64
_QOFF = 96
_KOFF = 112


def _body(x_ref, wbig_ref, bbig_ref, rep2_ref, wexp_ref, shift2_ref,
          bd_ref, biasT_ref, mask_ref, o_ref):
    f32 = jnp.float32
    bf16 = jnp.bfloat16
    x = x_ref[...].astype(bf16)
    qkv = jnp.dot(x, wbig_ref[...], preferred_element_type=f32) + bbig_ref[...]
    qkv = qkv.astype(bf16)

    rows = _BT * _N
    mask = mask_ref[...]

    zs = []
    for g in range(_GROUPS):
        r0 = g * rows
        zg = None
        for h in range(_H):
            q = qkv[r0:r0 + rows, _QOFF + 8 * h:_QOFF + 8 * h + 8]
            k = qkv[r0:r0 + rows, _KOFF + 8 * h:_KOFF + 8 * h + 8]
            va = qkv[r0:r0 + rows, 32 * h:32 * h + 32]
            s = lax.dot_general(q, k, (((1,), (1,)), ((), ())),
                                preferred_element_type=f32)
            p = jnp.exp(s).astype(bf16) * mask
            oa = jnp.dot(p, va, preferred_element_type=f32)
            t = oa[:, :_IMG] * pl.reciprocal(oa[:, _IMG:_IMG + 1],
                                             approx=True)
            zg = t if zg is None else zg + t
        zs.append(zg)

    v0 = qkv[:, _VRAW:_VRAW + 32].reshape(_BSTEP, _N, 32)[:, :_IMG, :]
    v0 = v0.reshape(_BSTEP * _IMG, 32)
    v0 = v0 * jnp.clip(v0 + 3.0, 0.0, 6.0)
    lhs = jnp.dot(v0, rep2_ref[...], preferred_element_type=f32)
    lhs = lhs.astype(bf16) * wexp_ref[...]
    conv = jnp.dot(lhs, shift2_ref[...],
                   preferred_element_type=f32)

    outs = []
    for g in range(_GROUPS):
        cg = conv[g * _BT * _IMG:(g + 1) * _BT * _IMG]
        cat = jnp.concatenate([zs[g], cg[:, :_IMG], cg[:, _IMG:]],
                              axis=0).astype(bf16)
        outs.append(jnp.dot(bd_ref[...], cat,
                            preferred_element_type=f32))
    outT = jnp.concatenate(outs, axis=0) + biasT_ref[...]
    o_ref[...] = outT.reshape(_BSTEP, _C, _IMG).transpose(0, 2, 1)


@jax.jit
def kernel(x, w_q, w_k, w_v, b_q, b_k, b_v, w_proj, w_exp, rep_mat,
           shift_stack, w_out, out_bias):
    B, N, C = x.shape
    f32 = jnp.float32
    bf16 = jnp.bfloat16

    wv0p = w_v[0] @ w_proj[0]
    wv1p = w_v[1] @ w_proj[1]
    bv0p = (b_v[0] @ w_proj[0])[0]
    bv1p = (b_v[1] @ w_proj[1])[0]
    zc16 = jnp.zeros((C, 16), f32)
    one16 = jnp.ones((16,), f32)
    wbig = jnp.concatenate([wv0p, zc16, wv1p, zc16, w_v[0], w_v[1],
                            w_q[0], w_q[1], w_k[0], w_k[1]],
                           axis=1).astype(bf16)
    bbig = jnp.concatenate([bv0p, one16, bv1p, one16, b_v[0, 0], b_v[1, 0],
                            b_q[0, 0], b_q[1, 0], b_k[0, 0],
                            b_k[1, 0]])[None, :]

    eye2 = jnp.eye(2, dtype=f32)
    rep2 = jnp.kron(eye2, rep_mat).astype(bf16)
    shift2 = jnp.kron(eye2, shift_stack).astype(bf16)
    wexp = jnp.tile(jnp.concatenate([w_exp[0], w_exp[1]], axis=1) * (1.0 / 6.0),
                    (_BSTEP, 1)).astype(bf16)

    woutT = w_out.T
    eyeb = jnp.eye(_BT, dtype=f32)
    bd = jnp.concatenate([jnp.kron(eyeb, woutT),
                          jnp.kron(eyeb, woutT[:, :_IMG]),
                          jnp.kron(eyeb, woutT[:, _IMG:])],
                         axis=1).astype(bf16)
    biasT = jnp.tile(out_bias.T, (_BSTEP, 1))

    rows = _BT * _N
    bi = jnp.arange(rows, dtype=jnp.int32) // _N
    mask = (bi[:, None] == bi[None, :]).astype(bf16)

    x2 = x.reshape(B * N, C)
    steps = B // _BSTEP
    const = lambda g: (0, 0)
    out = pl.pallas_call(
        _body,
        out_shape=jax.ShapeDtypeStruct((B, _IMG, _C), f32),
        grid=(steps,),
        in_specs=[
            pl.BlockSpec((_BSTEP * _N, C), lambda g: (g, 0)),
            pl.BlockSpec(wbig.shape, const),
            pl.BlockSpec(bbig.shape, const),
            pl.BlockSpec(rep2.shape, const),
            pl.BlockSpec(wexp.shape, const),
            pl.BlockSpec(shift2.shape, const),
            pl.BlockSpec(bd.shape, const),
            pl.BlockSpec(biasT.shape, const),
            pl.BlockSpec(mask.shape, const),
        ],
        out_specs=pl.BlockSpec((_BSTEP, _IMG, _C), lambda g: (g, 0, 0)),
        compiler_params=pltpu.CompilerParams(
            dimension_semantics=("parallel",)),
    )(x2, wbig, bbig, rep2, wexp, shift2, bd, biasT, mask)
    return out
